# baseline (speedup 1.0000x reference)
import functools

import jax
import jax.numpy as jnp
from jax import lax
from jax.experimental import pallas as pl
from jax.experimental.pallas import tpu as pltpu
from jax.experimental.pallas import tpu_sc as plsc

D = 128
CHUNK = 128
NC = 2
NS = 16
NW = NC * NS


@functools.lru_cache(maxsize=None)
def _make_gather(n_chunks: int):
    b_per_w = n_chunks * CHUNK
    b_pad = NW * b_per_w
    mesh = plsc.VectorSubcoreMesh(
        core_axis_name="c", subcore_axis_name="s",
        num_cores=NC, num_subcores=NS,
    )

    @functools.partial(
        pl.kernel,
        mesh=mesh,
        out_type=jax.ShapeDtypeStruct((b_pad, D), jnp.float32),
        scratch_types=[
            pltpu.VMEM((n_chunks, CHUNK), jnp.int32),
            pltpu.VMEM((CHUNK, D), jnp.float32),
            pltpu.VMEM((CHUNK, D), jnp.float32),
            pltpu.SemaphoreType.DMA,
            pltpu.SemaphoreType.DMA,
            pltpu.SemaphoreType.DMA,
            pltpu.SemaphoreType.DMA,
        ],
    )
    def gather_kernel(table, idx, out, idx_v, rows0, rows1, sg0, sg1, sw0, sw1):
        wid = lax.axis_index("s") * NC + lax.axis_index("c")
        base = wid * b_per_w
        rows = (rows0, rows1)
        sg = (sg0, sg1)
        sw = (sw0, sw1)

        pltpu.sync_copy(idx.at[wid], idx_v)

        def start_gather(c, b):
            cp = pltpu.make_async_copy(table.at[idx_v.at[c]], rows[b], sg[b])
            cp.start()
            return cp

        def start_write(c, b):
            cp = pltpu.make_async_copy(
                rows[b], out.at[pl.ds(base + c * CHUNK, CHUNK)], sw[b])
            cp.start()
            return cp

        writes = [None, None]
        g_prev = start_gather(0, 0)
        for c in range(1, n_chunks):
            b = c % 2
            if writes[b] is not None:
                writes[b].wait()
            g_cur = start_gather(c, b)
            g_prev.wait()
            writes[1 - b] = start_write(c - 1, 1 - b)
            g_prev = g_cur
        last = (n_chunks - 1) % 2
        g_prev.wait()
        w_last = start_write(n_chunks - 1, last)
        if writes[1 - last] is not None:
            writes[1 - last].wait()
        w_last.wait()

    return gather_kernel, b_pad


def kernel(graph, feat, select_idx):
    idx = select_idx.astype(jnp.int32)
    b = idx.shape[0]
    n_chunks = -(-b // (NW * CHUNK))
    fn, b_pad = _make_gather(n_chunks)
    idx_p = jnp.concatenate([idx, jnp.zeros((b_pad - b,), jnp.int32)])
    out = fn(feat, idx_p.reshape(NW, n_chunks, CHUNK))
    return out[:b]


# speedup vs baseline: 1.0213x; 1.0213x over previous
import functools

import jax
import jax.numpy as jnp
from jax import lax
from jax.experimental import pallas as pl
from jax.experimental.pallas import tpu as pltpu
from jax.experimental.pallas import tpu_sc as plsc

D = 128
CHUNK = 128
NC = 2
NS = 16
NW = NC * NS


@functools.lru_cache(maxsize=None)
def _make_gather(n_chunks: int):
    b_per_w = n_chunks * CHUNK
    b_pad = NW * b_per_w
    mesh = plsc.VectorSubcoreMesh(
        core_axis_name="c", subcore_axis_name="s",
        num_cores=NC, num_subcores=NS,
    )

    nbuf = min(6, n_chunks)

    @functools.partial(
        pl.kernel,
        mesh=mesh,
        out_type=jax.ShapeDtypeStruct((b_pad, D), jnp.float32),
        scratch_types=[
            pltpu.VMEM((n_chunks, CHUNK), jnp.int32)]
            + [pltpu.VMEM((CHUNK, D), jnp.float32)] * nbuf
            + [pltpu.SemaphoreType.DMA] * (2 * nbuf),
    )
    def gather_kernel(table, idx, out, idx_v, *bufs_sems):
        rows = bufs_sems[:nbuf]
        sg = bufs_sems[nbuf:2 * nbuf]
        sw = bufs_sems[2 * nbuf:]
        wid = lax.axis_index("s") * NC + lax.axis_index("c")
        base = wid * b_per_w

        pltpu.sync_copy(idx.at[wid], idx_v)

        def start_gather(c, b):
            cp = pltpu.make_async_copy(table.at[idx_v.at[c]], rows[b], sg[b])
            cp.start()
            return cp

        def start_write(c, b):
            cp = pltpu.make_async_copy(
                rows[b], out.at[pl.ds(base + c * CHUNK, CHUNK)], sw[b])
            cp.start()
            return cp

        gathers = [None] * nbuf
        writes = [None] * nbuf
        for c in range(nbuf - 1):
            gathers[c] = start_gather(c, c)
        for c in range(n_chunks):
            b = c % nbuf
            nxt = c + nbuf - 1
            if nxt < n_chunks:
                bn = nxt % nbuf
                if writes[bn] is not None:
                    writes[bn].wait()
                gathers[bn] = start_gather(nxt, bn)
            gathers[b].wait()
            writes[b] = start_write(c, b)
        for c in range(max(0, n_chunks - nbuf), n_chunks):
            writes[c % nbuf].wait()

    return gather_kernel, b_pad


def kernel(graph, feat, select_idx):
    idx = select_idx.astype(jnp.int32)
    b = idx.shape[0]
    n_chunks = -(-b // (NW * CHUNK))
    fn, b_pad = _make_gather(n_chunks)
    idx_p = jnp.concatenate([idx, jnp.zeros((b_pad - b,), jnp.int32)])
    out = fn(feat, idx_p.reshape(NW, n_chunks, CHUNK))
    return out[:b]


# speedup vs baseline: 4.8992x; 4.7970x over previous
import functools

import jax
import jax.numpy as jnp
from jax import lax
from jax.experimental import pallas as pl
from jax.experimental.pallas import tpu as pltpu
from jax.experimental.pallas import tpu_sc as plsc

D = 128
CHUNK = 128
NC = 2
NS = 16
NW = NC * NS


@functools.lru_cache(maxsize=None)
def _make_gather(b: int, n_slots: int, nbuf: int):
    n_full = b // CHUNK
    n_chunks = -(-b // CHUNK)
    mesh = plsc.VectorSubcoreMesh(
        core_axis_name="c", subcore_axis_name="s",
        num_cores=NC, num_subcores=NS,
    )

    @functools.partial(
        pl.kernel,
        mesh=mesh,
        out_type=jax.ShapeDtypeStruct((b, D), jnp.float32),
        scratch_types=[
            pltpu.VMEM((n_slots, CHUNK), jnp.int32)]
            + [pltpu.VMEM((CHUNK, D), jnp.float32)] * nbuf
            + [pltpu.SemaphoreType.DMA]
            + [pltpu.SemaphoreType.DMA] * (2 * nbuf),
    )
    def gather_kernel(table, idx, out, idx_v, *bufs_sems):
        rows = bufs_sems[:nbuf]
        si = bufs_sems[nbuf]
        sg = bufs_sems[nbuf + 1:nbuf + 1 + nbuf]
        sw = bufs_sems[nbuf + 1 + nbuf:]
        wid = lax.axis_index("s") * NC + lax.axis_index("c")

        def chunk_start(c):
            ci = wid * n_slots + c
            return jnp.where(
                ci < n_full, ci * CHUNK,
                jnp.where(ci == n_full, b - CHUNK,
                          (ci - n_chunks) * CHUNK)).astype(jnp.int32)

        starts = [chunk_start(c) for c in range(n_slots)]

        idx_cps = []
        for c in range(n_slots):
            cp = pltpu.make_async_copy(
                idx.at[pl.ds(starts[c], CHUNK)], idx_v.at[c], si)
            cp.start()
            idx_cps.append(cp)
        for cp in idx_cps:
            cp.wait()

        def start_gather(c, bf):
            cp = pltpu.make_async_copy(table.at[idx_v.at[c]], rows[bf], sg[bf])
            cp.start()
            return cp

        def start_write(c, bf):
            cp = pltpu.make_async_copy(
                rows[bf], out.at[pl.ds(starts[c], CHUNK)], sw[bf])
            cp.start()
            return cp

        gathers = [None] * nbuf
        writes = [None] * nbuf
        for c in range(nbuf - 1):
            gathers[c] = start_gather(c, c)
        for c in range(n_slots):
            bf = c % nbuf
            nxt = c + nbuf - 1
            if nxt < n_slots:
                bn = nxt % nbuf
                if writes[bn] is not None:
                    writes[bn].wait()
                gathers[bn] = start_gather(nxt, bn)
            gathers[bf].wait()
            writes[bf] = start_write(c, bf)
        for c in range(max(0, n_slots - nbuf), n_slots):
            writes[c % nbuf].wait()

    return gather_kernel


def kernel(graph, feat, select_idx):
    idx = select_idx.astype(jnp.int32)
    b = idx.shape[0]
    n_slots = -(-(-(-b // CHUNK)) // NW)
    fn = _make_gather(b, n_slots, 6)
    return fn(feat, idx)
